# speedup vs baseline: 4.2636x; 4.2636x over previous
import functools

import jax
import jax.numpy as jnp
import numpy as np
from jax import lax
from jax.experimental import pallas as pl
from jax.experimental.pallas import tpu as pltpu
from jax.experimental.pallas import tpu_sc as plsc

_VOCAB = 100000
_EMBED = 128
_BATCH = 1024
_SEQLEN = 200

_NW = 32
_ROWS_PER_W = (_BATCH * _SEQLEN) // _NW
_SEQ_PER_W = _ROWS_PER_W // _SEQLEN
_HALF = _SEQLEN // 2


def _positional_encoding_np():
    pos = np.arange(_SEQLEN, dtype=np.float32)[:, None]
    div = np.exp(
        np.arange(0, _EMBED, 2, dtype=np.float32) * (-np.log(10000.0) / _EMBED)
    )
    ang = pos * div[None, :]
    pe = np.zeros((_SEQLEN, _EMBED), dtype=np.float32)
    pe[:, 0::2] = np.sin(ang)
    pe[:, 1::2] = np.cos(ang)
    return pe


_PE = _positional_encoding_np()


def _sc_kernel(table_hbm, idx_hbm, pe_hbm, out_hbm, idx_v, pe_v, buf, sem):
    nc = 2
    wid = lax.axis_index("s") * nc + lax.axis_index("c")
    row_base = wid * _ROWS_PER_W

    idx_row0 = wid * (_ROWS_PER_W // _HALF)
    pltpu.sync_copy(idx_hbm.at[pl.ds(idx_row0, _ROWS_PER_W // _HALF)], idx_v)
    pltpu.sync_copy(pe_hbm, pe_v)

    def seq_body(s, _):
        cp0 = pltpu.async_copy(
            table_hbm.at[idx_v.at[2 * s]], buf.at[pl.ds(0, _HALF)], sem
        )
        cp1 = pltpu.async_copy(
            table_hbm.at[idx_v.at[2 * s + 1]], buf.at[pl.ds(_HALF, _HALF)], sem
        )
        cp0.wait()
        cp1.wait()

        def add_body(r, _):
            for k in range(_EMBED // 16):
                plsc.addupdate(
                    buf.at[r, pl.ds(k * 16, 16)],
                    pe_v[r, pl.ds(k * 16, 16)],
                )
            return 0

        lax.fori_loop(0, _SEQLEN, add_body, 0)

        pltpu.sync_copy(
            buf, out_hbm.at[pl.ds(row_base + s * _SEQLEN, _SEQLEN)]
        )
        return 0

    lax.fori_loop(0, _SEQ_PER_W, seq_body, 0)


@jax.jit
def _run(sequence_flat2d, token_table, pe):
    mesh = plsc.VectorSubcoreMesh(core_axis_name="c", subcore_axis_name="s")
    return pl.kernel(
        _sc_kernel,
        mesh=mesh,
        out_type=jax.ShapeDtypeStruct((_BATCH * _SEQLEN, _EMBED), jnp.float32),
        scratch_types=[
            pltpu.VMEM((_ROWS_PER_W // _HALF, _HALF), jnp.int32),
            pltpu.VMEM((_SEQLEN, _EMBED), jnp.float32),
            pltpu.VMEM((_SEQLEN, _EMBED), jnp.float32),
            pltpu.SemaphoreType.DMA,
        ],
    )(token_table, sequence_flat2d, pe)


def kernel(sequence, token_table):
    idx = sequence.reshape(-1).astype(jnp.int32).reshape(-1, _HALF)
    pe = jnp.asarray(_PE)
    out = _run(idx, token_table, pe)
    return out.reshape(_BATCH, _SEQLEN, _EMBED)


# speedup vs baseline: 6.4033x; 1.5019x over previous
import jax
import jax.numpy as jnp
import numpy as np
from jax import lax
from jax.experimental import pallas as pl
from jax.experimental.pallas import tpu as pltpu
from jax.experimental.pallas import tpu_sc as plsc

_VOCAB = 100000
_EMBED = 128
_BATCH = 1024
_SEQLEN = 200

_NW = 32
_ROWS_PER_W = (_BATCH * _SEQLEN) // _NW
_HALF = _SEQLEN // 2
_NCHUNK = _ROWS_PER_W // _SEQLEN
_NIDX = _ROWS_PER_W // _HALF


def _positional_encoding_np():
    pos = np.arange(_SEQLEN, dtype=np.float32)[:, None]
    div = np.exp(
        np.arange(0, _EMBED, 2, dtype=np.float32) * (-np.log(10000.0) / _EMBED)
    )
    ang = pos * div[None, :]
    pe = np.zeros((_SEQLEN, _EMBED), dtype=np.float32)
    pe[:, 0::2] = np.sin(ang)
    pe[:, 1::2] = np.cos(ang)
    return pe


_PE = _positional_encoding_np()


def _sc_kernel(table_hbm, idx_hbm, pe_hbm, out_hbm,
               idx_v, pe_v, b0, b1, g0, g1, o0, o1):
    bufs = (b0, b1)
    gsem = (g0, g1)
    osem = (o0, o1)

    nc = 2
    wid = lax.axis_index("s") * nc + lax.axis_index("c")
    row_base = wid * _ROWS_PER_W

    pltpu.sync_copy(idx_hbm.at[pl.ds(wid * _NIDX, _NIDX)], idx_v)
    pltpu.sync_copy(pe_hbm, pe_v)

    def issue_gather(c, b):
        pltpu.async_copy(table_hbm.at[idx_v.at[2 * c]],
                         bufs[b].at[pl.ds(0, _HALF)], gsem[b])
        pltpu.async_copy(table_hbm.at[idx_v.at[2 * c + 1]],
                         bufs[b].at[pl.ds(_HALF, _HALF)], gsem[b])

    def wait_gather(c, b):
        pltpu.make_async_copy(table_hbm.at[idx_v.at[2 * c]],
                              bufs[b].at[pl.ds(0, _HALF)], gsem[b]).wait()
        pltpu.make_async_copy(table_hbm.at[idx_v.at[2 * c + 1]],
                              bufs[b].at[pl.ds(_HALF, _HALF)], gsem[b]).wait()

    def issue_out(c, b):
        pltpu.async_copy(
            bufs[b], out_hbm.at[pl.ds(row_base + c * _SEQLEN, _SEQLEN)],
            osem[b])

    def wait_out(c, b):
        pltpu.make_async_copy(
            bufs[b], out_hbm.at[pl.ds(row_base + c * _SEQLEN, _SEQLEN)],
            osem[b]).wait()

    def pe_add(b):
        def body(r, _):
            for k in range(_EMBED // 16):
                plsc.addupdate(
                    bufs[b].at[r, pl.ds(k * 16, 16)],
                    pe_v[r, pl.ds(k * 16, 16)],
                )
            return 0

        lax.fori_loop(0, _SEQLEN, body, 0)

    issue_gather(0, 0)
    issue_gather(1, 1)
    wait_gather(0, 0)
    pe_add(0)
    issue_out(0, 0)

    def round_body(r, _):
        for j in range(2):
            s = 1 + 2 * r + j
            bg = j % 2
            bc = (1 + j) % 2
            wait_out(s - 1, bg)
            issue_gather(s + 1, bg)
            wait_gather(s, bc)
            pe_add(bc)
            issue_out(s, bc)
        return 0

    lax.fori_loop(0, (_NCHUNK - 2) // 2, round_body, 0)

    wait_gather(_NCHUNK - 1, 1)
    pe_add(1)
    issue_out(_NCHUNK - 1, 1)
    wait_out(_NCHUNK - 2, 0)
    wait_out(_NCHUNK - 1, 1)


@jax.jit
def _run(sequence_flat2d, token_table, pe):
    mesh = plsc.VectorSubcoreMesh(core_axis_name="c", subcore_axis_name="s")
    return pl.kernel(
        _sc_kernel,
        mesh=mesh,
        out_type=jax.ShapeDtypeStruct((_BATCH * _SEQLEN, _EMBED), jnp.float32),
        scratch_types=[
            pltpu.VMEM((_NIDX, _HALF), jnp.int32),
            pltpu.VMEM((_SEQLEN, _EMBED), jnp.float32),
            pltpu.VMEM((_SEQLEN, _EMBED), jnp.float32),
            pltpu.VMEM((_SEQLEN, _EMBED), jnp.float32),
            pltpu.SemaphoreType.DMA,
            pltpu.SemaphoreType.DMA,
            pltpu.SemaphoreType.DMA,
            pltpu.SemaphoreType.DMA,
        ],
    )(token_table, sequence_flat2d, pe)


def kernel(sequence, token_table):
    idx = sequence.reshape(-1).astype(jnp.int32).reshape(-1, _HALF)
    pe = jnp.asarray(_PE)
    out = _run(idx, token_table, pe)
    return out.reshape(_BATCH, _SEQLEN, _EMBED)
